# baseline (speedup 1.0000x reference)
import jax
import jax.numpy as jnp
from jax import lax
from jax.experimental import pallas as pl
from jax.experimental.pallas import tpu as pltpu
from jax.experimental.pallas import tpu_sc as plsc

_INFO = plsc.get_sparse_core_info()
_NC = _INFO.num_cores
_NS = _INFO.num_subcores
_NW = _NC * _NS

_BATCH = 16384
_DIM = 64
_WIDE = 128
_BLK = 128
_BLK_PER_W = _BATCH // (_NW * _BLK)
_PER_W = _BLK_PER_W * _BLK


_NTC = 100096 // _BLK
_COLS_PER_W = -(-_NTC // _NW)


def _widen_body(t2_hbm, t3_hbm, wide_hbm, slab_v, blk_v):
    wid = lax.axis_index("s") * _NC + lax.axis_index("c")
    rvecs = [lax.iota(jnp.int32, 16) + q * 16 for q in range(8)]

    def do_col(j, carry):
        cid = j * _NW + wid

        @pl.when(cid < _NTC - 1)
        def _():
            pltpu.sync_copy(t2_hbm.at[:, pl.ds(cid * _BLK, _BLK)], slab_v)
            for d in range(_DIM):
                dvec = jnp.full((16,), d, jnp.int32)
                for q in range(8):
                    v = slab_v[d, pl.ds(q * 16, 16)]
                    plsc.store_scatter(blk_v, [rvecs[q], dvec], v)
            pltpu.sync_copy(blk_v, wide_hbm.at[pl.ds(cid * _BLK, _BLK), :])

        return carry

    lax.fori_loop(0, _COLS_PER_W, do_col, 0)

    @pl.when(wid == _NW - 1)
    def _tail():
        pltpu.sync_copy(t3_hbm, slab_v)
        for d in range(_DIM):
            dvec = jnp.full((16,), d, jnp.int32)
            for q in range(8):
                v = slab_v[d, pl.ds(q * 16, 16)]
                plsc.store_scatter(blk_v, [rvecs[q], dvec], v)
        pltpu.sync_copy(
            blk_v, wide_hbm.at[pl.ds(100000 - _BLK, _BLK), :])


@jax.jit
def _widen(t2, t3):
    mesh = plsc.VectorSubcoreMesh(core_axis_name="c", subcore_axis_name="s")
    run = pl.kernel(
        _widen_body,
        out_type=jax.ShapeDtypeStruct((100000, _WIDE), jnp.float32),
        mesh=mesh,
        scratch_types=[
            pltpu.VMEM((_DIM, _BLK), jnp.float32),
            pltpu.VMEM((_BLK, _WIDE), jnp.float32),
        ],
        compiler_params=pltpu.CompilerParams(
            use_tc_tiling_on_sc=True,
            disable_bounds_checks=True,
            needs_layout_passes=False,
        ),
    )
    return run(t2, t3)


def _gather_body(idx_hbm, wide_hbm, out_hbm, idx_v, rows_v, sems):
    wid = lax.axis_index("s") * _NC + lax.axis_index("c")
    base = wid * _PER_W

    pltpu.sync_copy(idx_hbm.at[pl.ds(base, _PER_W)], idx_v)
    copies = [
        pltpu.async_copy(
            wide_hbm.at[idx_v.at[pl.ds(k * _BLK, _BLK)]],
            rows_v.at[k],
            sems.at[k],
        )
        for k in range(_BLK_PER_W)
    ]
    for k in range(_BLK_PER_W):
        copies[k].wait()
        pltpu.sync_copy(
            rows_v.at[k], out_hbm.at[pl.ds(base + k * _BLK, _BLK), :])


@jax.jit
def _gather(idx, wide):
    mesh = plsc.VectorSubcoreMesh(core_axis_name="c", subcore_axis_name="s")
    run = pl.kernel(
        _gather_body,
        out_type=jax.ShapeDtypeStruct((_BATCH, _WIDE), jnp.float32),
        mesh=mesh,
        scratch_types=[
            pltpu.VMEM((_PER_W,), jnp.int32),
            pltpu.VMEM((_BLK_PER_W, _BLK, _WIDE), jnp.float32),
            pltpu.SemaphoreType.DMA((_BLK_PER_W,)),
        ],
        compiler_params=pltpu.CompilerParams(
            use_tc_tiling_on_sc=True,
            disable_bounds_checks=True,
        ),
    )
    return run(idx, wide)


def kernel(indices, latents):
    idx = indices.astype(jnp.int32)
    table = latents.reshape(latents.shape[0], _DIM)
    t2 = table.T
    wide = _widen(t2, t2[:, 100000 - _BLK:])
    out128 = _gather(idx, wide)
    return out128[:, :_DIM].reshape(_BATCH, 1, _DIM)


# speedup vs baseline: 2.3439x; 2.3439x over previous
import jax
import jax.numpy as jnp
from jax import lax
from jax.experimental import pallas as pl
from jax.experimental.pallas import tpu as pltpu
from jax.experimental.pallas import tpu_sc as plsc

_INFO = plsc.get_sparse_core_info()
_NC = _INFO.num_cores
_NS = _INFO.num_subcores
_NW = _NC * _NS

_BATCH = 16384
_DIM = 64
_WIDE = 128
_BLK = 128
_BLK_PER_W = _BATCH // (_NW * _BLK)
_PER_W = _BLK_PER_W * _BLK


def _gather_body(idx_hbm, wide_hbm, out_hbm, idx_v, rows_v, sems):
    wid = lax.axis_index("s") * _NC + lax.axis_index("c")
    base = wid * _PER_W

    pltpu.sync_copy(idx_hbm.at[pl.ds(base, _PER_W)], idx_v)
    copies = [
        pltpu.async_copy(
            wide_hbm.at[idx_v.at[pl.ds(k * _BLK, _BLK)]],
            rows_v.at[k],
            sems.at[k],
        )
        for k in range(_BLK_PER_W)
    ]
    for k in range(_BLK_PER_W):
        copies[k].wait()
        pltpu.sync_copy(
            rows_v.at[k], out_hbm.at[pl.ds(base + k * _BLK, _BLK), :])


@jax.jit
def _gather(idx, wide):
    mesh = plsc.VectorSubcoreMesh(core_axis_name="c", subcore_axis_name="s")
    run = pl.kernel(
        _gather_body,
        out_type=jax.ShapeDtypeStruct((_BATCH, _WIDE), jnp.float32),
        mesh=mesh,
        scratch_types=[
            pltpu.VMEM((_PER_W,), jnp.int32),
            pltpu.VMEM((_BLK_PER_W, _BLK, _WIDE), jnp.float32),
            pltpu.SemaphoreType.DMA((_BLK_PER_W,)),
        ],
        compiler_params=pltpu.CompilerParams(
            use_tc_tiling_on_sc=True,
            disable_bounds_checks=True,
        ),
    )
    return run(idx, wide)


def kernel(indices, latents):
    idx = indices.astype(jnp.int32)
    wide = jnp.pad(
        latents, ((0, 0), (0, 0), (0, _WIDE - _DIM))
    ).reshape(latents.shape[0], _WIDE)
    out128 = _gather(idx, wide)
    return out128[:, :_DIM].reshape(_BATCH, 1, _DIM)
